# baseline (speedup 1.0000x reference)
import functools

import jax
import jax.numpy as jnp
from jax import lax
from jax.experimental import pallas as pl
from jax.experimental.pallas import tpu as pltpu
from jax.experimental.pallas import tpu_sc as plsc

KC = 8192
D = 32
BR = 256

NC = 2
NS = 16
NW = NC * NS
GCH = 128


def _vq_tc_body(z_ref, w_ref, a2_ref, b2_ref, idx_ref):
    z = z_ref[...]
    w = w_ref[...]
    a2 = a2_ref[...]
    b2 = b2_ref[...]
    dot = lax.dot_general(
        z, w, (((1,), (1,)), ((), ())),
        precision=lax.Precision.HIGHEST,
        preferred_element_type=jnp.float32,
    )
    d2 = jnp.maximum((a2 + b2) - 2.0 * dot, 0.0)
    pd = jnp.where(d2 == 0.0, 0.0, d2 * lax.rsqrt(d2))
    minv = jnp.min(pd, axis=1, keepdims=True)
    iota = lax.broadcasted_iota(jnp.int32, pd.shape, 1)
    idx = jnp.min(jnp.where(pd <= minv, iota, KC), axis=1)
    idx_ref[...] = idx.reshape(1, 1, BR)


def _tc_indices(flat, weight, a2, b2, n):
    grid = n // BR
    idx3 = pl.pallas_call(
        _vq_tc_body,
        grid=(grid,),
        in_specs=[
            pl.BlockSpec((BR, D), lambda i: (i, 0)),
            pl.BlockSpec((KC, D), lambda i: (0, 0)),
            pl.BlockSpec((BR, 1), lambda i: (i, 0)),
            pl.BlockSpec((1, KC), lambda i: (0, 0)),
        ],
        out_specs=pl.BlockSpec((1, 1, BR), lambda i: (i, 0, 0)),
        out_shape=jax.ShapeDtypeStruct((grid, 1, BR), jnp.int32),
    )(flat, weight, a2, b2)
    return idx3.reshape(n)


def _sc_gather(weight, idx_flat, n):
    chunks_per_w = n // (NW * GCH)
    idx2 = idx_flat.reshape(n // GCH, GCH)
    mesh = plsc.VectorSubcoreMesh(core_axis_name="c", subcore_axis_name="s")

    @functools.partial(
        pl.kernel,
        out_type=jax.ShapeDtypeStruct((n, D), jnp.float32),
        mesh=mesh,
        compiler_params=pltpu.CompilerParams(use_tc_tiling_on_sc=False),
        scratch_types=[
            pltpu.VMEM((chunks_per_w, GCH), jnp.int32),
            pltpu.VMEM((GCH, D), jnp.float32),
            pltpu.SemaphoreType.DMA,
        ],
    )
    def gather_kernel(w_hbm, idx_hbm, out_hbm, idx_v, rows_v, sem):
        wid = lax.axis_index("s") * NC + lax.axis_index("c")
        row0 = wid * chunks_per_w
        pltpu.sync_copy(idx_hbm.at[pl.ds(row0, chunks_per_w)], idx_v)
        for j in range(chunks_per_w):
            pltpu.async_copy(w_hbm.at[idx_v.at[j]], rows_v, sem).wait()
            pltpu.sync_copy(rows_v, out_hbm.at[pl.ds((row0 + j) * GCH, GCH)])

    return gather_kernel(weight, idx2)


def kernel(z_e, weight):
    B, V, C = z_e.shape
    n = B * V
    flat = z_e.reshape(n, C)
    a2 = jnp.sum(flat * flat, axis=1, keepdims=True)
    b2 = jnp.sum(weight * weight, axis=1)[None, :]
    idx_flat = _tc_indices(flat, weight, a2, b2, n)
    z_q = _sc_gather(weight, idx_flat, n).reshape(z_e.shape)
    z_q_st = z_e + (z_q - z_e)
    return (z_q_st, z_q, idx_flat.reshape(B, V))


# speedup vs baseline: 1.0948x; 1.0948x over previous
import functools

import jax
import jax.numpy as jnp
from jax import lax
from jax.experimental import pallas as pl
from jax.experimental.pallas import tpu as pltpu
from jax.experimental.pallas import tpu_sc as plsc

KC = 8192
D = 32
BR = 256

NC = 2
NS = 16
NW = NC * NS
GCH = 128


def _vq_tc_body(z_ref, w_ref, b2h_ref, idx_ref):
    z = z_ref[...]
    w = w_ref[...]
    dot = lax.dot_general(
        z, w, (((1,), (1,)), ((), ())),
        precision=lax.Precision.HIGHEST,
        preferred_element_type=jnp.float32,
    )
    s = b2h_ref[...] - dot
    minv = jnp.min(s, axis=1, keepdims=True)
    iota = lax.broadcasted_iota(jnp.int32, s.shape, 1)
    idx = jnp.min(jnp.where(s <= minv, iota, KC), axis=1)
    idx_ref[...] = idx.reshape(1, 1, BR)


def _tc_indices(flat, weight, b2h, n):
    grid = n // BR
    idx3 = pl.pallas_call(
        _vq_tc_body,
        grid=(grid,),
        in_specs=[
            pl.BlockSpec((BR, D), lambda i: (i, 0)),
            pl.BlockSpec((KC, D), lambda i: (0, 0)),
            pl.BlockSpec((1, KC), lambda i: (0, 0)),
        ],
        out_specs=pl.BlockSpec((1, 1, BR), lambda i: (i, 0, 0)),
        out_shape=jax.ShapeDtypeStruct((grid, 1, BR), jnp.int32),
    )(flat, weight, b2h)
    return idx3.reshape(n)


def _sc_gather(weight, idx_flat, n):
    chunks_per_w = n // (NW * GCH)
    idx2 = idx_flat.reshape(n // GCH, GCH)
    mesh = plsc.VectorSubcoreMesh(core_axis_name="c", subcore_axis_name="s")

    @functools.partial(
        pl.kernel,
        out_type=jax.ShapeDtypeStruct((n, D), jnp.float32),
        mesh=mesh,
        compiler_params=pltpu.CompilerParams(use_tc_tiling_on_sc=False),
        scratch_types=[
            pltpu.VMEM((chunks_per_w, GCH), jnp.int32),
            pltpu.VMEM((GCH, D), jnp.float32),
            pltpu.SemaphoreType.DMA,
        ],
    )
    def gather_kernel(w_hbm, idx_hbm, out_hbm, idx_v, rows_v, sem):
        wid = lax.axis_index("s") * NC + lax.axis_index("c")
        row0 = wid * chunks_per_w
        pltpu.sync_copy(idx_hbm.at[pl.ds(row0, chunks_per_w)], idx_v)
        for j in range(chunks_per_w):
            pltpu.async_copy(w_hbm.at[idx_v.at[j]], rows_v, sem).wait()
            pltpu.sync_copy(rows_v, out_hbm.at[pl.ds((row0 + j) * GCH, GCH)])

    return gather_kernel(weight, idx2)


def kernel(z_e, weight):
    B, V, C = z_e.shape
    n = B * V
    flat = z_e.reshape(n, C)
    b2h = (0.5 * jnp.sum(weight * weight, axis=1))[None, :]
    idx_flat = _tc_indices(flat, weight, b2h, n)
    z_q = _sc_gather(weight, idx_flat, n).reshape(z_e.shape)
    z_q_st = z_e + (z_q - z_e)
    return (z_q_st, z_q, idx_flat.reshape(B, V))


# speedup vs baseline: 1.0973x; 1.0022x over previous
import functools

import jax
import jax.numpy as jnp
from jax import lax
from jax.experimental import pallas as pl
from jax.experimental.pallas import tpu as pltpu
from jax.experimental.pallas import tpu_sc as plsc

KC = 8192
D = 32
BR = 512

NC = 2
NS = 16
NW = NC * NS
GCH = 128


def _vq_tc_body(z_ref, w_ref, b2h_ref, idx_ref):
    z = z_ref[...]
    w = w_ref[...]
    dot = lax.dot_general(
        z, w, (((1,), (1,)), ((), ())),
        precision=lax.Precision.HIGHEST,
        preferred_element_type=jnp.float32,
    )
    s = b2h_ref[...] - dot
    minv = jnp.min(s, axis=1, keepdims=True)
    iota = lax.broadcasted_iota(jnp.int32, s.shape, 1)
    idx = jnp.min(jnp.where(s <= minv, iota, KC), axis=1)
    idx_ref[...] = idx.reshape(1, 1, BR)


def _tc_indices(flat, weight, b2h, n):
    grid = n // BR
    idx3 = pl.pallas_call(
        _vq_tc_body,
        grid=(grid,),
        in_specs=[
            pl.BlockSpec((BR, D), lambda i: (i, 0)),
            pl.BlockSpec((KC, D), lambda i: (0, 0)),
            pl.BlockSpec((1, KC), lambda i: (0, 0)),
        ],
        out_specs=pl.BlockSpec((1, 1, BR), lambda i: (i, 0, 0)),
        out_shape=jax.ShapeDtypeStruct((grid, 1, BR), jnp.int32),
    )(flat, weight, b2h)
    return idx3.reshape(n)


def _sc_gather(weight, idx_flat, n):
    chunks_per_w = n // (NW * GCH)
    idx2 = idx_flat.reshape(n // GCH, GCH)
    mesh = plsc.VectorSubcoreMesh(core_axis_name="c", subcore_axis_name="s")

    @functools.partial(
        pl.kernel,
        out_type=jax.ShapeDtypeStruct((n, D), jnp.float32),
        mesh=mesh,
        compiler_params=pltpu.CompilerParams(use_tc_tiling_on_sc=False),
        scratch_types=[
            pltpu.VMEM((chunks_per_w, GCH), jnp.int32),
            pltpu.VMEM((GCH, D), jnp.float32),
            pltpu.SemaphoreType.DMA,
        ],
    )
    def gather_kernel(w_hbm, idx_hbm, out_hbm, idx_v, rows_v, sem):
        wid = lax.axis_index("s") * NC + lax.axis_index("c")
        row0 = wid * chunks_per_w
        pltpu.sync_copy(idx_hbm.at[pl.ds(row0, chunks_per_w)], idx_v)
        for j in range(chunks_per_w):
            pltpu.async_copy(w_hbm.at[idx_v.at[j]], rows_v, sem).wait()
            pltpu.sync_copy(rows_v, out_hbm.at[pl.ds((row0 + j) * GCH, GCH)])

    return gather_kernel(weight, idx2)


def kernel(z_e, weight):
    B, V, C = z_e.shape
    n = B * V
    flat = z_e.reshape(n, C)
    b2h = (0.5 * jnp.sum(weight * weight, axis=1))[None, :]
    idx_flat = _tc_indices(flat, weight, b2h, n)
    z_q = _sc_gather(weight, idx_flat, n).reshape(z_e.shape)
    z_q_st = z_e + (z_q - z_e)
    return (z_q_st, z_q, idx_flat.reshape(B, V))
